# baseline (speedup 1.0000x reference)
import functools
import jax
import jax.numpy as jnp
import numpy as np
from jax import lax
from jax.experimental import pallas as pl
from jax.experimental.pallas import tpu as pltpu

NUM_CLASSES_K = 1000
EPS_K = 0.1
RATIO_K = 0.6
BATCH_K = 16384
BM = 2048
NB = BATCH_K // BM
SPB = 4
SUB = BM // SPB
ROWS = SUB // 128
TOPK = int(BATCH_K * RATIO_K)
MININT = np.int32(-2147483648)
MAXPOS = np.int32(2147483647)


def _ce_kernel(x_ref, t_ref, o_ref, ps_ref):
    i = pl.program_id(0)
    j = pl.program_id(1)
    x = x_ref[pl.ds(j * SUB, SUB), :]
    t = t_ref[0, 0, :]
    m = jnp.max(x, axis=1)
    se = jnp.sum(jnp.exp(x - m[:, None]), axis=1)
    lse = m + jnp.log(se)
    cols = lax.broadcasted_iota(jnp.int32, x.shape, 1)
    w_hi = (1.0 - EPS_K) + EPS_K / NUM_CLASSES_K
    w_lo = EPS_K / NUM_CLASSES_K
    r = jnp.sum(x * jnp.where(cols == t[:, None], w_hi, w_lo), axis=1)
    ps = (lse - r) / NUM_CLASSES_K
    ps_ref[pl.ds((i * SPB + j) * ROWS, ROWS), :] = ps.reshape(ROWS, 128)

    @pl.when((i == NB - 1) & (j == SPB - 1))
    def _epilogue():
        v = ps_ref[...]
        b = lax.bitcast_convert_type(v, jnp.int32)
        skey = b ^ (jnp.right_shift(b, 31) & MAXPOS)

        def body(tstep, p):
            bit = jnp.left_shift(jnp.int32(1), 31 - tstep)
            cand = p | bit
            cnt = jnp.sum((skey >= (cand ^ MININT)).astype(jnp.int32))
            return jnp.where(cnt >= TOPK, cand, p)

        p = lax.fori_loop(0, 32, body, jnp.int32(0))
        skey_k = p ^ MININT
        bk = jnp.where(skey_k >= 0, skey_k, skey_k ^ MAXPOS)
        v_k = lax.bitcast_convert_type(bk, jnp.float32)
        gt = skey > skey_k
        cnt_gt = jnp.sum(gt.astype(jnp.int32))
        sum_gt = jnp.sum(jnp.where(gt, v, 0.0))
        loss = (sum_gt + (TOPK - cnt_gt).astype(jnp.float32) * v_k) / TOPK
        o_ref[...] = loss.reshape(1, 1)


@jax.jit
def kernel(inputs, targets):
    t3 = targets.astype(jnp.int32).reshape(NB * SPB, 1, SUB)
    out = pl.pallas_call(
        _ce_kernel,
        grid=(NB, SPB),
        in_specs=[
            pl.BlockSpec((BM, NUM_CLASSES_K), lambda i, j: (i, 0)),
            pl.BlockSpec((1, 1, SUB), lambda i, j: (i * SPB + j, 0, 0)),
        ],
        out_specs=pl.BlockSpec((1, 1), lambda i, j: (0, 0)),
        out_shape=jax.ShapeDtypeStruct((1, 1), jnp.float32),
        scratch_shapes=[pltpu.VMEM((128, 128), jnp.float32)],
        compiler_params=pltpu.CompilerParams(
            dimension_semantics=("arbitrary", "arbitrary"),
        ),
    )(inputs, t3)
    return out[0, 0]


# speedup vs baseline: 1.2313x; 1.2313x over previous
import functools
import jax
import jax.numpy as jnp
import numpy as np
from jax import lax
from jax.experimental import pallas as pl
from jax.experimental.pallas import tpu as pltpu

NUM_CLASSES_K = 1000
EPS_K = 0.1
RATIO_K = 0.6
BATCH_K = 16384
BM = 2048
NB = BATCH_K // BM
ROWS = BM // 128
TOPK = int(BATCH_K * RATIO_K)
MININT = np.int32(-2147483648)
MAXPOS = np.int32(2147483647)


def _ce_kernel(x_ref, t_ref, o_ref, ps_ref):
    i = pl.program_id(0)
    x = x_ref[...]
    t = t_ref[0, 0, :]
    w_hi = (1.0 - EPS_K) + EPS_K / NUM_CLASSES_K
    w_lo = EPS_K / NUM_CLASSES_K
    t2 = t[:, None]
    m_acc = jnp.full((BM, 128), -jnp.inf, jnp.float32)
    r_acc = jnp.zeros((BM, 128), jnp.float32)
    for off in range(0, NUM_CLASSES_K, 128):
        w = min(128, NUM_CLASSES_K - off)
        xc = x[:, off:off + w]
        ci = lax.broadcasted_iota(jnp.int32, (BM, w), 1) + off
        wc = jnp.where(ci == t2, w_hi, w_lo)
        if w == 128:
            m_acc = jnp.maximum(m_acc, xc)
            r_acc = r_acc + xc * wc
        else:
            m_acc = jnp.concatenate(
                [jnp.maximum(m_acc[:, :w], xc), m_acc[:, w:]], axis=1)
            r_acc = jnp.concatenate(
                [r_acc[:, :w] + xc * wc, r_acc[:, w:]], axis=1)
    m = jnp.max(m_acc, axis=1)
    r = jnp.sum(r_acc, axis=1)
    se = jnp.sum(jnp.exp(x - m[:, None]), axis=1)
    lse = m + jnp.log(se)
    ps = (lse - r) / NUM_CLASSES_K
    ps_ref[pl.ds(i * ROWS, ROWS), :] = ps.reshape(ROWS, 128)

    @pl.when(i == NB - 1)
    def _epilogue():
        v = ps_ref[...]
        b = lax.bitcast_convert_type(v, jnp.int32)
        skey = b ^ (jnp.right_shift(b, 31) & MAXPOS)

        def body(tstep, p):
            sh = 30 - 2 * tstep
            c1 = p | jnp.left_shift(jnp.int32(1), sh)
            c2 = p | jnp.left_shift(jnp.int32(2), sh)
            c3 = p | jnp.left_shift(jnp.int32(3), sh)
            n1 = jnp.sum((skey >= (c1 ^ MININT)).astype(jnp.int32))
            n2 = jnp.sum((skey >= (c2 ^ MININT)).astype(jnp.int32))
            n3 = jnp.sum((skey >= (c3 ^ MININT)).astype(jnp.int32))
            return jnp.where(
                n3 >= TOPK, c3,
                jnp.where(n2 >= TOPK, c2, jnp.where(n1 >= TOPK, c1, p)))

        p = lax.fori_loop(0, 16, body, jnp.int32(0))
        skey_k = p ^ MININT
        bk = jnp.where(skey_k >= 0, skey_k, skey_k ^ MAXPOS)
        v_k = lax.bitcast_convert_type(bk, jnp.float32)
        gt = skey > skey_k
        cnt_gt = jnp.sum(gt.astype(jnp.int32))
        sum_gt = jnp.sum(jnp.where(gt, v, 0.0))
        loss = (sum_gt + (TOPK - cnt_gt).astype(jnp.float32) * v_k) / TOPK
        o_ref[...] = loss.reshape(1, 1)


@jax.jit
def kernel(inputs, targets):
    t3 = targets.astype(jnp.int32).reshape(NB, 1, BM)
    out = pl.pallas_call(
        _ce_kernel,
        grid=(NB,),
        in_specs=[
            pl.BlockSpec((BM, NUM_CLASSES_K), lambda i: (i, 0)),
            pl.BlockSpec((1, 1, BM), lambda i: (i, 0, 0)),
        ],
        out_specs=pl.BlockSpec((1, 1), lambda i: (0, 0)),
        out_shape=jax.ShapeDtypeStruct((1, 1), jnp.float32),
        scratch_shapes=[pltpu.VMEM((128, 128), jnp.float32)],
        compiler_params=pltpu.CompilerParams(
            dimension_semantics=("arbitrary",),
        ),
    )(inputs, t3)
    return out[0, 0]


# speedup vs baseline: 1.2343x; 1.0024x over previous
import jax
import jax.numpy as jnp
import numpy as np
from jax import lax
from jax.experimental import pallas as pl
from jax.experimental.pallas import tpu as pltpu

NUM_CLASSES_K = 1000
EPS_K = 0.1
RATIO_K = 0.6
BATCH_K = 16384
BM = 2048
NB = BATCH_K // BM
ROWS = BM // 128
TOPK = int(BATCH_K * RATIO_K)
MININT = np.int32(-2147483648)
MAXPOS = np.int32(2147483647)


def _ce_kernel(x_ref, t_ref, o_ref, ps_ref):
    i = pl.program_id(0)
    x = x_ref[...]
    t = t_ref[0, 0, :]
    w_hi = (1.0 - EPS_K) + EPS_K / NUM_CLASSES_K
    w_lo = EPS_K / NUM_CLASSES_K
    t2 = t[:, None]
    m_acc = jnp.full((BM, 128), -jnp.inf, jnp.float32)
    r_acc = jnp.zeros((BM, 128), jnp.float32)
    for off in range(0, NUM_CLASSES_K, 128):
        w = min(128, NUM_CLASSES_K - off)
        xc = x[:, off:off + w]
        ci = lax.broadcasted_iota(jnp.int32, (BM, w), 1) + off
        wc = jnp.where(ci == t2, w_hi, w_lo)
        if w == 128:
            m_acc = jnp.maximum(m_acc, xc)
            r_acc = r_acc + xc * wc
        else:
            m_acc = jnp.concatenate(
                [jnp.maximum(m_acc[:, :w], xc), m_acc[:, w:]], axis=1)
            r_acc = jnp.concatenate(
                [r_acc[:, :w] + xc * wc, r_acc[:, w:]], axis=1)
    m = jnp.max(m_acc, axis=1)
    r = jnp.sum(r_acc, axis=1)
    se = jnp.sum(jnp.exp(x - m[:, None]), axis=1)
    lse = m + jnp.log(se)
    ps = (lse - r) / NUM_CLASSES_K
    ps_ref[pl.ds(i * ROWS, ROWS), :] = ps.reshape(ROWS, 128)

    @pl.when(i == NB - 1)
    def _epilogue():
        v = ps_ref[...]
        b = lax.bitcast_convert_type(v, jnp.int32)
        skey = b ^ (jnp.right_shift(b, 31) & MAXPOS)

        def body(tstep, p):
            sh = 30 - 2 * tstep
            c1 = p | jnp.left_shift(jnp.int32(1), sh)
            c2 = p | jnp.left_shift(jnp.int32(2), sh)
            c3 = p | jnp.left_shift(jnp.int32(3), sh)
            n1 = jnp.sum((skey >= (c1 ^ MININT)).astype(jnp.int32))
            n2 = jnp.sum((skey >= (c2 ^ MININT)).astype(jnp.int32))
            n3 = jnp.sum((skey >= (c3 ^ MININT)).astype(jnp.int32))
            return jnp.where(
                n3 >= TOPK, c3,
                jnp.where(n2 >= TOPK, c2, jnp.where(n1 >= TOPK, c1, p)))

        p = lax.fori_loop(0, 16, body, jnp.int32(0))
        skey_k = p ^ MININT
        bk = jnp.where(skey_k >= 0, skey_k, skey_k ^ MAXPOS)
        v_k = lax.bitcast_convert_type(bk, jnp.float32)
        gt = skey > skey_k
        cnt_gt = jnp.sum(gt.astype(jnp.int32))
        sum_gt = jnp.sum(jnp.where(gt, v, 0.0))
        loss = (sum_gt + (TOPK - cnt_gt).astype(jnp.float32) * v_k) / TOPK
        o_ref[...] = loss.reshape(1, 1)


@jax.jit
def kernel(inputs, targets):
    t3 = targets.astype(jnp.int32).reshape(NB, 1, BM)
    out = pl.pallas_call(
        _ce_kernel,
        grid=(NB,),
        in_specs=[
            pl.BlockSpec((BM, NUM_CLASSES_K), lambda i: (i, 0)),
            pl.BlockSpec((1, 1, BM), lambda i: (i, 0, 0)),
        ],
        out_specs=pl.BlockSpec((1, 1), lambda i: (0, 0)),
        out_shape=jax.ShapeDtypeStruct((1, 1), jnp.float32),
        scratch_shapes=[pltpu.VMEM((128, 128), jnp.float32)],
        compiler_params=pltpu.CompilerParams(
            dimension_semantics=("arbitrary",),
        ),
    )(inputs, t3)
    return out[0, 0]


# speedup vs baseline: 1.2412x; 1.0056x over previous
import jax
import jax.numpy as jnp
import numpy as np
from jax import lax
from jax.experimental import pallas as pl
from jax.experimental.pallas import tpu as pltpu

NUM_CLASSES_K = 1000
EPS_K = 0.1
RATIO_K = 0.6
BATCH_K = 16384
BM = 2048
NB = BATCH_K // BM
ROWS = BM // 128
TOPK = int(BATCH_K * RATIO_K)
MININT = np.int32(-2147483648)
MAXPOS = np.int32(2147483647)


def _ce_kernel(x_ref, t_ref, o_ref, ps_ref):
    i = pl.program_id(0)
    t = t_ref[0, 0, :]
    w_hi = (1.0 - EPS_K) + EPS_K / NUM_CLASSES_K
    w_lo = EPS_K / NUM_CLASSES_K
    t2 = t[:, None]
    m_acc = jnp.full((BM, 128), -jnp.inf, jnp.float32)
    r_acc = jnp.zeros((BM, 128), jnp.float32)
    for off in range(0, NUM_CLASSES_K, 128):
        w = min(128, NUM_CLASSES_K - off)
        xc = x_ref[:, off:off + w]
        ci = lax.broadcasted_iota(jnp.int32, (BM, w), 1) + off
        wc = jnp.where(ci == t2, w_hi, w_lo)
        if w == 128:
            m_acc = jnp.maximum(m_acc, xc)
            r_acc = r_acc + xc * wc
        else:
            m_acc = jnp.concatenate(
                [jnp.maximum(m_acc[:, :w], xc), m_acc[:, w:]], axis=1)
            r_acc = jnp.concatenate(
                [r_acc[:, :w] + xc * wc, r_acc[:, w:]], axis=1)
    m = jnp.max(m_acc, axis=1)
    r = jnp.sum(r_acc, axis=1)
    se = jnp.sum(jnp.exp(x_ref[...] - m[:, None]), axis=1)
    lse = m + jnp.log(se)
    ps = (lse - r) / NUM_CLASSES_K
    ps_ref[pl.ds(i * ROWS, ROWS), :] = ps.reshape(ROWS, 128)

    @pl.when(i == NB - 1)
    def _epilogue():
        v = ps_ref[...]
        b = lax.bitcast_convert_type(v, jnp.int32)
        skey = b ^ (jnp.right_shift(b, 31) & MAXPOS)

        def body(tstep, p):
            sh = 30 - 2 * tstep
            c1 = p | jnp.left_shift(jnp.int32(1), sh)
            c2 = p | jnp.left_shift(jnp.int32(2), sh)
            c3 = p | jnp.left_shift(jnp.int32(3), sh)
            n1 = jnp.sum((skey >= (c1 ^ MININT)).astype(jnp.int32))
            n2 = jnp.sum((skey >= (c2 ^ MININT)).astype(jnp.int32))
            n3 = jnp.sum((skey >= (c3 ^ MININT)).astype(jnp.int32))
            return jnp.where(
                n3 >= TOPK, c3,
                jnp.where(n2 >= TOPK, c2, jnp.where(n1 >= TOPK, c1, p)))

        p = lax.fori_loop(0, 16, body, jnp.int32(0))
        skey_k = p ^ MININT
        bk = jnp.where(skey_k >= 0, skey_k, skey_k ^ MAXPOS)
        v_k = lax.bitcast_convert_type(bk, jnp.float32)
        gt = skey > skey_k
        cnt_gt = jnp.sum(gt.astype(jnp.int32))
        sum_gt = jnp.sum(jnp.where(gt, v, 0.0))
        loss = (sum_gt + (TOPK - cnt_gt).astype(jnp.float32) * v_k) / TOPK
        o_ref[...] = loss.reshape(1, 1)


@jax.jit
def kernel(inputs, targets):
    t3 = targets.astype(jnp.int32).reshape(NB, 1, BM)
    out = pl.pallas_call(
        _ce_kernel,
        grid=(NB,),
        in_specs=[
            pl.BlockSpec((BM, NUM_CLASSES_K), lambda i: (i, 0)),
            pl.BlockSpec((1, 1, BM), lambda i: (i, 0, 0)),
        ],
        out_specs=pl.BlockSpec((1, 1), lambda i: (0, 0)),
        out_shape=jax.ShapeDtypeStruct((1, 1), jnp.float32),
        scratch_shapes=[pltpu.VMEM((128, 128), jnp.float32)],
        compiler_params=pltpu.CompilerParams(
            dimension_semantics=("arbitrary",),
        ),
    )(inputs, t3)
    return out[0, 0]
